# baseline (speedup 1.0000x reference)
import jax
import jax.numpy as jnp
import numpy as np
from jax.experimental import pallas as pl

S_SAMPLE = 1620
K_NN = 32


def _fps(xyz, s):
    N = xyz.shape[0]

    def body(i, state):
        idxs, dists, last = state
        d = jnp.sum((xyz - xyz[last]) ** 2, axis=-1)
        dists = jnp.minimum(dists, d)
        nxt = jnp.argmax(dists).astype(jnp.int32)
        idxs = idxs.at[i].set(nxt)
        return (idxs, dists, nxt)

    idxs0 = jnp.zeros((s,), dtype=jnp.int32)
    dists0 = jnp.full((N,), 1e10, dtype=xyz.dtype)
    idxs, dists, last = jax.lax.fori_loop(1, s, body, (idxs0, dists0, jnp.int32(0)))
    return idxs


def _knn_group(s, k, coords, features):
    fps_idx = jax.lax.stop_gradient(jax.vmap(lambda c: _fps(c, s))(coords))
    new_coords = jnp.take_along_axis(coords, fps_idx[:, :, None], axis=1)
    new_feat = jnp.take_along_axis(features, fps_idx[:, :, None], axis=1)
    d2 = (jnp.sum(new_coords ** 2, axis=-1, keepdims=True)
          - 2.0 * jnp.einsum('bsd,bnd->bsn', new_coords, coords)
          + jnp.sum(coords ** 2, axis=-1)[:, None, :])
    idx = jnp.argsort(d2, axis=-1)[:, :, :k]
    grouped = jax.vmap(lambda f, i: f[i])(features, idx)
    centered = grouped - new_feat[:, :, None, :]
    tiled = jnp.broadcast_to(new_feat[:, :, None, :], grouped.shape)
    return jnp.concatenate([centered, tiled], axis=-1)


def _bn_stats(h, gamma, beta, eps=1e-5):
    mean = jnp.mean(h, axis=(0, 2), keepdims=True)
    var = jnp.var(h, axis=(0, 2), keepdims=True)
    scale = gamma[None, :, None] / jnp.sqrt(var + eps)
    shift = beta[None, :, None] - mean * scale
    return scale, shift


def _relu_maxpool_kernel(h_ref, o_ref):
    o_ref[...] = jnp.max(jax.nn.relu(h_ref[...]), axis=2)


def kernel(x, coords, W1, W2, gamma1, beta1, gamma2, beta2):
    feats = jnp.transpose(x, (0, 2, 1))
    nf = _knn_group(S_SAMPLE, K_NN, coords, feats)
    b, s, k, d = nf.shape
    nf = jnp.transpose(nf, (0, 1, 3, 2)).reshape(-1, d, k)
    h = jnp.einsum('od,bdk->bok', W1, nf)
    sc1, sh1 = _bn_stats(h, gamma1, beta1)
    h = jax.nn.relu(sc1 * h + sh1)
    h = jnp.einsum('oc,bck->bok', W2, h)
    sc2, sh2 = _bn_stats(h, gamma2, beta2)
    h = sc2 * h + sh2

    rows = b * s
    blk = 720
    out = pl.pallas_call(
        _relu_maxpool_kernel,
        grid=(rows // blk,),
        in_specs=[pl.BlockSpec((blk, h.shape[1], h.shape[2]), lambda i: (i, 0, 0))],
        out_specs=pl.BlockSpec((blk, h.shape[1]), lambda i: (i, 0)),
        out_shape=jax.ShapeDtypeStruct((rows, h.shape[1]), h.dtype),
    )(h)
    return jnp.transpose(out.reshape(b, s, -1), (0, 2, 1))


# speedup vs baseline: 7.2245x; 7.2245x over previous
import jax
import jax.numpy as jnp
import numpy as np
from jax.experimental import pallas as pl
from jax.experimental.pallas import tpu as pltpu

S_SAMPLE = 1620
K_NN = 32
_S_PAD = 1664
_S_BLK = 208
_Q_BLK = 416
_EPS = 1e-5
_HI = jax.lax.Precision.HIGHEST


def _fps_kernel(cx_ref, cy_ref, cz_ref, idx_ref):
    X = cx_ref[...]; Y = cy_ref[...]; Z = cz_ref[...]
    b, n = X.shape
    s = idx_ref.shape[1]
    iota = jax.lax.broadcasted_iota(jnp.int32, (b, n), 1)
    lane = jax.lax.broadcasted_iota(jnp.int32, (b, 128), 1)

    def body(i, carry):
        dists, lx, ly, lz, acc = carry
        dx = X - lx; dy = Y - ly; dz = Z - lz
        sx = dx * dx; sy = dy * dy; sz = dz * dz
        d = (sx + sy) + sz
        dists = jnp.minimum(dists, d)
        m = jnp.max(dists, axis=1, keepdims=True)
        sel = dists == m
        nxt = jnp.min(jnp.where(sel, iota, n), axis=1, keepdims=True)
        one = iota == nxt
        lx = jnp.sum(jnp.where(one, X, 0.0), axis=1, keepdims=True)
        ly = jnp.sum(jnp.where(one, Y, 0.0), axis=1, keepdims=True)
        lz = jnp.sum(jnp.where(one, Z, 0.0), axis=1, keepdims=True)
        acc = jnp.where(lane == (i % 128), nxt, acc)

        @pl.when(i % 128 == 127)
        def _():
            base = pl.multiple_of((i // 128) * 128, 128)
            idx_ref[:, pl.ds(base, 128)] = acc

        return (dists, lx, ly, lz, acc)

    carry0 = (jnp.full((b, n), 1e10, jnp.float32),
              X[:, 0:1], Y[:, 0:1], Z[:, 0:1],
              jnp.zeros((b, 128), jnp.int32))
    final = jax.lax.fori_loop(1, S_SAMPLE, body, carry0)
    idx_ref[:, s - 128:] = final[4]


def _fps_pallas(coords, s_pad):
    b = coords.shape[0]
    ct = jnp.transpose(coords, (2, 0, 1))
    return pl.pallas_call(
        _fps_kernel,
        out_shape=jax.ShapeDtypeStruct((b, s_pad), jnp.int32),
    )(ct[0], ct[1], ct[2])


def _knn_p_kernel(q_ref, c_ref, f_ref, w_ref, idx_ref, p_ref, d2_ref):
    q = q_ref[0]
    qx = q[:, 0:1]; qy = q[:, 1:2]; qz = q[:, 2:3]
    X = c_ref[0, 0:1, :]
    Y = c_ref[0, 1:2, :]
    Z = c_ref[0, 2:3, :]
    xn = X * X + Y * Y + Z * Z
    qn = qx * qx + qy * qy + qz * qz
    e = jax.lax.dot_general(q[:, 0:3], c_ref[0], (((1,), (0,)), ((), ())),
                            preferred_element_type=jnp.float32)
    d2_ref[...] = (qn - 2.0 * e) + xn
    n = d2_ref.shape[1]
    iota = jax.lax.broadcasted_iota(jnp.int32, d2_ref.shape, 1)
    inf = jnp.float32(jnp.inf)
    for j in range(idx_ref.shape[2]):
        d2 = d2_ref[...]
        m = jnp.min(d2, axis=1, keepdims=True)
        idxv = jnp.min(jnp.where(d2 == m, iota, n), axis=1, keepdims=True)
        idx_ref[0, :, j:j + 1] = idxv
        d2_ref[...] = jnp.where(iota == idxv, inf, d2)
    p_ref[0] = jax.lax.dot_general(f_ref[0], w_ref[...], (((1,), (0,)), ((), ())),
                                   precision=_HI,
                                   preferred_element_type=jnp.float32)


def _knn_p_pallas(newc_pad, coords, feats, w1at, k):
    b, s_pad, _ = newc_pad.shape
    n = coords.shape[1]
    d = feats.shape[2]
    c = w1at.shape[1]
    ct = jnp.transpose(coords, (0, 2, 1))
    ns = s_pad // _S_BLK
    nb = n // ns
    return pl.pallas_call(
        _knn_p_kernel,
        grid=(b, ns),
        in_specs=[
            pl.BlockSpec((1, _S_BLK, 3), lambda bi, j: (bi, j, 0)),
            pl.BlockSpec((1, 3, n), lambda bi, j: (bi, 0, 0)),
            pl.BlockSpec((1, nb, d), lambda bi, j: (bi, j, 0)),
            pl.BlockSpec((d, c), lambda bi, j: (0, 0)),
        ],
        out_specs=[
            pl.BlockSpec((1, _S_BLK, k), lambda bi, j: (bi, j, 0)),
            pl.BlockSpec((1, nb, c), lambda bi, j: (bi, j, 0)),
        ],
        out_shape=[
            jax.ShapeDtypeStruct((b, s_pad, k), jnp.int32),
            jax.ShapeDtypeStruct((b, n, c), jnp.float32),
        ],
        scratch_shapes=[pltpu.VMEM((_S_BLK, n), jnp.float32)],
    )(newc_pad, ct, feats, w1at)


def _stats1_kernel(h_ref, nf_ref, wq_ref, sums_ref):
    j = pl.program_id(0)
    c = h_ref.shape[1]
    h = h_ref[...].reshape(_Q_BLK, K_NN, c)
    Q = jax.lax.dot_general(nf_ref[...], wq_ref[...], (((1,), (0,)), ((), ())),
                            precision=_HI, preferred_element_type=jnp.float32)
    h = h + Q[:, None, :]
    g = j * _Q_BLK + jax.lax.broadcasted_iota(jnp.int32, (_Q_BLK, 1, 1), 0)
    valid = (g % _S_PAD) < S_SAMPLE
    hm = jnp.where(valid, h, 0.0).reshape(_Q_BLK * K_NN, c)
    s1 = jnp.sum(hm, axis=0, keepdims=True)
    s2 = jnp.sum(hm * hm, axis=0, keepdims=True)

    @pl.when(j == 0)
    def _():
        sums_ref[...] = jnp.zeros_like(sums_ref)

    sums_ref[0:1, :] += s1
    sums_ref[1:2, :] += s2


def _stats2_kernel(h_ref, nf_ref, wq_ref, aff1_ref, m2_ref, mu_ref):
    j = pl.program_id(0)
    c = h_ref.shape[1]
    h = h_ref[...].reshape(_Q_BLK, K_NN, c)
    Q = jax.lax.dot_general(nf_ref[...], wq_ref[...], (((1,), (0,)), ((), ())),
                            precision=_HI, preferred_element_type=jnp.float32)
    h = h + Q[:, None, :]
    a = jax.nn.relu(aff1_ref[0:1, :][None] * h + aff1_ref[1:2, :][None])
    g = j * _Q_BLK + jax.lax.broadcasted_iota(jnp.int32, (_Q_BLK, 1, 1), 0)
    valid = (g % _S_PAD) < S_SAMPLE
    am = jnp.where(valid, a, 0.0).reshape(_Q_BLK * K_NN, c)
    m2 = jax.lax.dot_general(am, am, (((0,), (0,)), ((), ())),
                             precision=_HI, preferred_element_type=jnp.float32)
    mu = jnp.sum(am, axis=0, keepdims=True)

    @pl.when(j == 0)
    def _():
        m2_ref[...] = jnp.zeros_like(m2_ref)
        mu_ref[...] = jnp.zeros_like(mu_ref)

    m2_ref[...] += m2
    mu_ref[0:1, :] += mu


def _final_kernel(h_ref, nf_ref, wq_ref, aff1_ref, w2t_ref, aff2_ref, o_ref):
    c = h_ref.shape[1]
    h = h_ref[...].reshape(_Q_BLK, K_NN, c)
    Q = jax.lax.dot_general(nf_ref[...], wq_ref[...], (((1,), (0,)), ((), ())),
                            precision=_HI, preferred_element_type=jnp.float32)
    h = h + Q[:, None, :]
    a = jax.nn.relu(aff1_ref[0:1, :][None] * h + aff1_ref[1:2, :][None])
    h2 = jax.lax.dot_general(a.reshape(_Q_BLK * K_NN, c), w2t_ref[...],
                             (((1,), (0,)), ((), ())),
                             precision=_HI, preferred_element_type=jnp.float32)
    y = jax.nn.relu(aff2_ref[0:1, :] * h2 + aff2_ref[1:2, :]).reshape(_Q_BLK, K_NN, c)
    o_ref[...] = jnp.max(y, axis=1)


def _mlp_pallas(h1raw, newfeat, wqt, W2, gamma1, beta1, gamma2, beta2, m_valid):
    nq, d = newfeat.shape
    c = W2.shape[0]
    ng = nq // _Q_BLK
    sums = pl.pallas_call(
        _stats1_kernel,
        grid=(ng,),
        in_specs=[
            pl.BlockSpec((_Q_BLK * K_NN, c), lambda j: (j, 0)),
            pl.BlockSpec((_Q_BLK, d), lambda j: (j, 0)),
            pl.BlockSpec((d, c), lambda j: (0, 0)),
        ],
        out_specs=pl.BlockSpec((8, c), lambda j: (0, 0)),
        out_shape=jax.ShapeDtypeStruct((8, c), jnp.float32),
    )(h1raw, newfeat, wqt)
    mean1 = sums[0] / m_valid
    var1 = sums[1] / m_valid - mean1 * mean1
    sc1 = gamma1 / jnp.sqrt(var1 + _EPS)
    sh1 = beta1 - mean1 * sc1
    aff1 = jnp.zeros((8, c), jnp.float32).at[0].set(sc1).at[1].set(sh1)

    m2, mu = pl.pallas_call(
        _stats2_kernel,
        grid=(ng,),
        in_specs=[
            pl.BlockSpec((_Q_BLK * K_NN, c), lambda j: (j, 0)),
            pl.BlockSpec((_Q_BLK, d), lambda j: (j, 0)),
            pl.BlockSpec((d, c), lambda j: (0, 0)),
            pl.BlockSpec((8, c), lambda j: (0, 0)),
        ],
        out_specs=[
            pl.BlockSpec((c, c), lambda j: (0, 0)),
            pl.BlockSpec((8, c), lambda j: (0, 0)),
        ],
        out_shape=[
            jax.ShapeDtypeStruct((c, c), jnp.float32),
            jax.ShapeDtypeStruct((8, c), jnp.float32),
        ],
    )(h1raw, newfeat, wqt, aff1)
    mu_a = mu[0] / m_valid
    mean2 = W2 @ mu_a
    e2 = jnp.sum((W2 @ (m2 / m_valid)) * W2, axis=1)
    var2 = e2 - mean2 * mean2
    sc2 = gamma2 / jnp.sqrt(var2 + _EPS)
    sh2 = beta2 - mean2 * sc2
    aff2 = jnp.zeros((8, c), jnp.float32).at[0].set(sc2).at[1].set(sh2)

    return pl.pallas_call(
        _final_kernel,
        grid=(ng,),
        in_specs=[
            pl.BlockSpec((_Q_BLK * K_NN, c), lambda j: (j, 0)),
            pl.BlockSpec((_Q_BLK, d), lambda j: (j, 0)),
            pl.BlockSpec((d, c), lambda j: (0, 0)),
            pl.BlockSpec((8, c), lambda j: (0, 0)),
            pl.BlockSpec((c, c), lambda j: (0, 0)),
            pl.BlockSpec((8, c), lambda j: (0, 0)),
        ],
        out_specs=pl.BlockSpec((_Q_BLK, c), lambda j: (j, 0)),
        out_shape=jax.ShapeDtypeStruct((nq, c), jnp.float32),
    )(h1raw, newfeat, wqt, aff1, W2.T, aff2)


def kernel(x, coords, W1, W2, gamma1, beta1, gamma2, beta2):
    b, d, n = x.shape
    c = W2.shape[0]
    feats = jnp.transpose(x, (0, 2, 1))
    fps_pad = _fps_pallas(coords, _S_PAD)
    newc_pad = jnp.take_along_axis(coords, fps_pad[:, :, None], axis=1)
    newf_pad = jnp.take_along_axis(feats, fps_pad[:, :, None], axis=1)
    W1a = W1[:, :d]
    W1b = W1[:, d:]
    idx, P = _knn_p_pallas(newc_pad, coords, feats, W1a.T, K_NN)
    h1raw = jax.vmap(lambda p, i: p[i])(P, idx).reshape(b * _S_PAD * K_NN, c)
    newfeat = newf_pad.reshape(b * _S_PAD, d)
    out = _mlp_pallas(h1raw, newfeat, (W1b - W1a).T, W2,
                      gamma1, beta1, gamma2, beta2, b * S_SAMPLE * K_NN)
    out = out.reshape(b, _S_PAD, c)[:, :S_SAMPLE]
    return jnp.transpose(out, (0, 2, 1))
